# Initial kernel scaffold; baseline (speedup 1.0000x reference)
import jax
import jax.numpy as jnp
from jax.experimental import pallas as pl


def kernel(x, edge_index, edge_weight, W1, b1, W2, b2):
    raise NotImplementedError("write your pallas kernel here")


# speedup vs baseline: 11.5700x; 11.5700x over previous
import functools

import jax
import jax.numpy as jnp
from jax import lax
from jax.experimental import pallas as pl
from jax.experimental.pallas import tpu as pltpu
from jax.experimental.pallas import tpu_sc as plsc

N = 10000
E = 320000
NPAD = 10240
NC = 2
NS = 16
NW = NC * NS
RPT = NPAD // NS
EPW = E // NW
CH = 80
NCHUNK = EPW // CH


def _sc_mesh():
    return plsc.VectorSubcoreMesh(core_axis_name="c", subcore_axis_name="s")


@functools.partial(
    pl.kernel,
    out_type=jax.ShapeDtypeStruct((NC, NPAD), jnp.float32),
    mesh=_sc_mesh(),
    scratch_types=[
        pltpu.VMEM((CH,), jnp.int32),
        pltpu.VMEM((CH,), jnp.float32),
        pltpu.VMEM_SHARED((NPAD,), jnp.float32),
    ],
)
def _sc_deg(dst, ew, zeros, out, dst_v, ew_v, acc):
    cid = lax.axis_index("c")
    sid = lax.axis_index("s")
    wid = cid * NS + sid
    pltpu.sync_copy(zeros.at[pl.ds(sid * RPT, RPT)], acc.at[pl.ds(sid * RPT, RPT)])
    plsc.subcore_barrier()
    base0 = wid * EPW

    def body(j, carry):
        off = base0 + j * CH
        pltpu.sync_copy(dst.at[pl.ds(off, CH)], dst_v)
        pltpu.sync_copy(ew.at[pl.ds(off, CH)], ew_v)
        pltpu.sync_copy(ew_v, acc.at[dst_v], add=True)
        return carry

    lax.fori_loop(0, NCHUNK, body, 0)
    plsc.subcore_barrier()
    pltpu.sync_copy(acc.at[pl.ds(sid * RPT, RPT)], out.at[cid, pl.ds(sid * RPT, RPT)])


def _make_sc_agg(D):
    @functools.partial(
        pl.kernel,
        out_type=jax.ShapeDtypeStruct((NC, NPAD, D), jnp.float32),
        mesh=_sc_mesh(),
        compiler_params=pltpu.CompilerParams(use_tc_tiling_on_sc=False),
        scratch_types=[
            pltpu.VMEM((CH,), jnp.int32),
            pltpu.VMEM((CH,), jnp.int32),
            pltpu.VMEM((CH,), jnp.float32),
            pltpu.VMEM((CH, D), jnp.float32),
            pltpu.VMEM_SHARED((NPAD, D), jnp.float32),
        ],
    )
    def _sc_agg(src, dst, ew, tbl, zeros, out, src_v, dst_v, ew_v, rows_v, acc):
        cid = lax.axis_index("c")
        sid = lax.axis_index("s")
        wid = cid * NS + sid
        pltpu.sync_copy(zeros.at[pl.ds(sid * RPT, RPT)], acc.at[pl.ds(sid * RPT, RPT)])
        plsc.subcore_barrier()
        base0 = wid * EPW

        def body(j, carry):
            off = base0 + j * CH
            pltpu.sync_copy(src.at[pl.ds(off, CH)], src_v)
            pltpu.sync_copy(dst.at[pl.ds(off, CH)], dst_v)
            pltpu.sync_copy(ew.at[pl.ds(off, CH)], ew_v)
            pltpu.sync_copy(tbl.at[src_v], rows_v)

            def scale(g, c2):
                ev = ew_v[pl.ds(g * 16, 16)]
                for i in range(16):
                    s = ev[i]
                    for k in range(D // 16):
                        sl = pl.ds(k * 16, 16)
                        rows_v[g * 16 + i, sl] = rows_v[g * 16 + i, sl] * s
                return c2

            lax.fori_loop(0, CH // 16, scale, 0)
            pltpu.sync_copy(rows_v, acc.at[dst_v], add=True)
            return carry

        lax.fori_loop(0, NCHUNK, body, 0)
        plsc.subcore_barrier()
        pltpu.sync_copy(
            acc.at[pl.ds(sid * RPT, RPT)], out.at[cid, pl.ds(sid * RPT, RPT)]
        )

    return _sc_agg


_sc_agg64 = _make_sc_agg(64)
_sc_agg32 = _make_sc_agg(32)


def _tc1_body(d0, d1, x, w, dinv_o, xws_o):
    deg = d0[...] + d1[...] + 1.0
    dinv = lax.rsqrt(deg)
    xw = jnp.dot(x[...], w[...], preferred_element_type=jnp.float32)
    dinv_o[...] = dinv
    xws_o[...] = xw * dinv


_tc1 = pl.pallas_call(
    _tc1_body,
    out_shape=[
        jax.ShapeDtypeStruct((N, 1), jnp.float32),
        jax.ShapeDtypeStruct((N, 64), jnp.float32),
    ],
)


def _tc2_body(a0, a1, xws, dinv, b1, w2, out):
    t = (a0[...] + a1[...] + xws[...]) * dinv[...] + b1[...]
    h = jnp.maximum(t, 0.0)
    hw2 = jnp.dot(h, w2[...], preferred_element_type=jnp.float32)
    out[...] = hw2 * dinv[...]


_tc2 = pl.pallas_call(
    _tc2_body,
    out_shape=jax.ShapeDtypeStruct((N, 32), jnp.float32),
)


def _tc3_body(c0, c1, hw, dinv, b2, out):
    out[...] = (c0[...] + c1[...] + hw[...]) * dinv[...] + b2[...]


_tc3 = pl.pallas_call(
    _tc3_body,
    out_shape=jax.ShapeDtypeStruct((N, 32), jnp.float32),
)


def kernel(x, edge_index, edge_weight, W1, b1, W2, b2):
    ei = edge_index.astype(jnp.int32)
    src = ei[0]
    dst = ei[1]
    ew = edge_weight
    z1 = jnp.zeros((NPAD,), jnp.float32)
    z64 = jnp.zeros((NPAD, 64), jnp.float32)
    z32 = jnp.zeros((NPAD, 32), jnp.float32)

    degp = _sc_deg(dst, ew, z1)
    d0 = degp[0, :N, None]
    d1 = degp[1, :N, None]
    dinv, xws1 = _tc1(d0, d1, x, W1)
    acc1 = _sc_agg64(src, dst, ew, xws1, z64)
    hw2s = _tc2(acc1[0, :N], acc1[1, :N], xws1, dinv,
                b1.reshape(1, -1), W2)
    acc2 = _sc_agg32(src, dst, ew, hw2s, z32)
    z = _tc3(acc2[0, :N], acc2[1, :N], hw2s, dinv, b2.reshape(1, -1))
    return z


# speedup vs baseline: 24.5660x; 2.1232x over previous
import functools

import jax
import jax.numpy as jnp
from jax import lax
from jax.experimental import pallas as pl
from jax.experimental.pallas import tpu as pltpu
from jax.experimental.pallas import tpu_sc as plsc

N = 10000
E = 320000
NPAD = 10240
NC = 2
NS = 16
NW = NC * NS
RPT = NPAD // NS
EPW = E // NW
CH = 80
NCHUNK = EPW // CH
DEG_LAG = 8


def _sc_mesh():
    return plsc.VectorSubcoreMesh(core_axis_name="c", subcore_axis_name="s")


@functools.partial(
    pl.kernel,
    out_type=jax.ShapeDtypeStruct((NC, NPAD), jnp.float32),
    mesh=_sc_mesh(),
    scratch_types=[
        pltpu.VMEM((NCHUNK, CH), jnp.int32),
        pltpu.VMEM((NCHUNK, CH), jnp.float32),
        pltpu.VMEM_SHARED((NPAD,), jnp.float32),
        pltpu.SemaphoreType.DMA,
    ],
)
def _sc_deg(dst3, ew3, zeros, out, dst_all, ew_all, acc, ssem):
    cid = lax.axis_index("c")
    sid = lax.axis_index("s")
    wid = cid * NS + sid
    pltpu.sync_copy(dst3.at[wid], dst_all)
    pltpu.sync_copy(ew3.at[wid], ew_all)
    pltpu.sync_copy(zeros.at[pl.ds(sid * RPT, RPT)], acc.at[pl.ds(sid * RPT, RPT)])
    plsc.subcore_barrier()

    def fire(j):
        pltpu.make_async_copy(ew_all.at[j], acc.at[dst_all.at[j]], ssem).start(
            add=True
        )

    def drain(j):
        pltpu.make_async_copy(ew_all.at[j], acc.at[dst_all.at[j]], ssem).wait()

    def body(j, carry):
        fire(j)

        @pl.when(j >= DEG_LAG)
        def _():
            drain(j - DEG_LAG)

        return carry

    lax.fori_loop(0, NCHUNK, body, 0)
    for j in range(NCHUNK - DEG_LAG, NCHUNK):
        drain(j)
    plsc.subcore_barrier()
    pltpu.sync_copy(acc.at[pl.ds(sid * RPT, RPT)], out.at[cid, pl.ds(sid * RPT, RPT)])


def _make_sc_agg(D):
    @functools.partial(
        pl.kernel,
        out_type=jax.ShapeDtypeStruct((NC, NPAD, D), jnp.float32),
        mesh=_sc_mesh(),
        compiler_params=pltpu.CompilerParams(use_tc_tiling_on_sc=False),
        scratch_types=[
            pltpu.VMEM((NCHUNK, CH), jnp.int32),
            pltpu.VMEM((NCHUNK, CH), jnp.int32),
            pltpu.VMEM((NCHUNK, CH), jnp.float32),
            pltpu.VMEM((CH, D), jnp.float32),
            pltpu.VMEM((CH, D), jnp.float32),
            pltpu.VMEM_SHARED((NPAD, D), jnp.float32),
            pltpu.SemaphoreType.DMA,
            pltpu.SemaphoreType.DMA,
            pltpu.SemaphoreType.DMA,
            pltpu.SemaphoreType.DMA,
        ],
    )
    def _sc_agg(src3, dst3, ew3, tbl, zeros, out, src_all, dst_all, ew_all,
                rows0, rows1, acc, gsem0, gsem1, ssem0, ssem1):
        cid = lax.axis_index("c")
        sid = lax.axis_index("s")
        wid = cid * NS + sid
        pltpu.sync_copy(src3.at[wid], src_all)
        pltpu.sync_copy(dst3.at[wid], dst_all)
        pltpu.sync_copy(ew3.at[wid], ew_all)
        pltpu.sync_copy(zeros.at[pl.ds(sid * RPT, RPT)], acc.at[pl.ds(sid * RPT, RPT)])
        plsc.subcore_barrier()

        def g_copy(j, rows, gsem):
            return pltpu.make_async_copy(tbl.at[src_all.at[j]], rows, gsem)

        def s_copy(j, rows, ssem):
            return pltpu.make_async_copy(rows, acc.at[dst_all.at[j]], ssem)

        def scale(rows, j):
            def grp(g, c):
                ev = ew_all[j, pl.ds(g * 16, 16)]
                for i in range(16):
                    s = ev[i]
                    for k in range(D // 16):
                        sl = pl.ds(k * 16, 16)
                        rows[g * 16 + i, sl] = rows[g * 16 + i, sl] * s
                return c

            lax.fori_loop(0, CH // 16, grp, 0)

        g_copy(0, rows0, gsem0).start()
        g_copy(1, rows1, gsem1).start()

        def pair(j2, carry):
            j = 2 * j2
            g_copy(j, rows0, gsem0).wait()
            scale(rows0, j)
            s_copy(j, rows0, ssem0).start(add=True)

            @pl.when(j + 2 < NCHUNK)
            def _():
                s_copy(j, rows0, ssem0).wait()
                g_copy(j + 2, rows0, gsem0).start()

            @pl.when(j + 1 < NCHUNK)
            def _():
                g_copy(j + 1, rows1, gsem1).wait()
                scale(rows1, j + 1)
                s_copy(j + 1, rows1, ssem1).start(add=True)

                @pl.when(j + 3 < NCHUNK)
                def _():
                    s_copy(j + 1, rows1, ssem1).wait()
                    g_copy(j + 3, rows1, gsem1).start()

            return carry

        lax.fori_loop(0, (NCHUNK + 1) // 2, pair, 0)
        s_copy(NCHUNK - 1, rows0, ssem0).wait()
        s_copy(NCHUNK - 2, rows1, ssem1).wait()
        plsc.subcore_barrier()
        pltpu.sync_copy(
            acc.at[pl.ds(sid * RPT, RPT)], out.at[cid, pl.ds(sid * RPT, RPT)]
        )

    return _sc_agg


_sc_agg64 = _make_sc_agg(64)
_sc_agg32 = _make_sc_agg(32)


def _tc1_body(d0, d1, x, w, dinv_o, xws_o):
    deg = d0[...] + d1[...] + 1.0
    dinv = lax.rsqrt(deg)
    xw = jnp.dot(x[...], w[...], preferred_element_type=jnp.float32)
    dinv_o[...] = dinv
    xws_o[...] = xw * dinv


_tc1 = pl.pallas_call(
    _tc1_body,
    out_shape=[
        jax.ShapeDtypeStruct((N, 1), jnp.float32),
        jax.ShapeDtypeStruct((N, 64), jnp.float32),
    ],
)


def _tc2_body(a0, a1, xws, dinv, b1, w2, out):
    t = (a0[...] + a1[...] + xws[...]) * dinv[...] + b1[...]
    h = jnp.maximum(t, 0.0)
    hw2 = jnp.dot(h, w2[...], preferred_element_type=jnp.float32)
    out[...] = hw2 * dinv[...]


_tc2 = pl.pallas_call(
    _tc2_body,
    out_shape=jax.ShapeDtypeStruct((N, 32), jnp.float32),
)


def _tc3_body(c0, c1, hw, dinv, b2, out):
    out[...] = (c0[...] + c1[...] + hw[...]) * dinv[...] + b2[...]


_tc3 = pl.pallas_call(
    _tc3_body,
    out_shape=jax.ShapeDtypeStruct((N, 32), jnp.float32),
)


def kernel(x, edge_index, edge_weight, W1, b1, W2, b2):
    ei = edge_index.astype(jnp.int32)
    src3 = ei[0].reshape(NW, NCHUNK, CH)
    dst3 = ei[1].reshape(NW, NCHUNK, CH)
    ew3 = edge_weight.reshape(NW, NCHUNK, CH)
    z1 = jnp.zeros((NPAD,), jnp.float32)
    z64 = jnp.zeros((NPAD, 64), jnp.float32)
    z32 = jnp.zeros((NPAD, 32), jnp.float32)

    degp = _sc_deg(dst3, ew3, z1)
    d0 = degp[0, :N, None]
    d1 = degp[1, :N, None]
    dinv, xws1 = _tc1(d0, d1, x, W1)
    acc1 = _sc_agg64(src3, dst3, ew3, xws1, z64)
    hw2s = _tc2(acc1[0, :N], acc1[1, :N], xws1, dinv,
                b1.reshape(1, -1), W2)
    acc2 = _sc_agg32(src3, dst3, ew3, hw2s, z32)
    z = _tc3(acc2[0, :N], acc2[1, :N], hw2s, dinv, b2.reshape(1, -1))
    return z


# speedup vs baseline: 26.1804x; 1.0657x over previous
import functools

import jax
import jax.numpy as jnp
from jax import lax
from jax.experimental import pallas as pl
from jax.experimental.pallas import tpu as pltpu
from jax.experimental.pallas import tpu_sc as plsc

N = 10000
E = 320000
NPAD = 10240
NC = 2
NS = 16
NW = NC * NS
RPT = NPAD // NS
EPW = E // NW
CH = 80
NCHUNK = EPW // CH
NBUF = 5
DEG_LAG = 8


def _sc_mesh():
    return plsc.VectorSubcoreMesh(core_axis_name="c", subcore_axis_name="s")


@functools.partial(
    pl.kernel,
    out_type=jax.ShapeDtypeStruct((NC, NPAD), jnp.float32),
    mesh=_sc_mesh(),
    scratch_types=[
        pltpu.VMEM((NCHUNK, CH), jnp.int32),
        pltpu.VMEM((NCHUNK, CH), jnp.float32),
        pltpu.VMEM_SHARED((NPAD,), jnp.float32),
        pltpu.SemaphoreType.DMA,
    ],
)
def _sc_deg(dst3, ew3, zeros, out, dst_all, ew_all, acc, ssem):
    cid = lax.axis_index("c")
    sid = lax.axis_index("s")
    wid = cid * NS + sid
    pltpu.sync_copy(dst3.at[wid], dst_all)
    pltpu.sync_copy(ew3.at[wid], ew_all)
    pltpu.sync_copy(zeros.at[pl.ds(sid * RPT, RPT)], acc.at[pl.ds(sid * RPT, RPT)])
    plsc.subcore_barrier()

    def fire(j):
        pltpu.make_async_copy(ew_all.at[j], acc.at[dst_all.at[j]], ssem).start(
            add=True
        )

    def drain(j):
        pltpu.make_async_copy(ew_all.at[j], acc.at[dst_all.at[j]], ssem).wait()

    def body(j, carry):
        fire(j)

        @pl.when(j >= DEG_LAG)
        def _():
            drain(j - DEG_LAG)

        return carry

    lax.fori_loop(0, NCHUNK, body, 0)
    for j in range(NCHUNK - DEG_LAG, NCHUNK):
        drain(j)
    plsc.subcore_barrier()
    pltpu.sync_copy(acc.at[pl.ds(sid * RPT, RPT)], out.at[cid, pl.ds(sid * RPT, RPT)])


def _make_sc_agg(D):
    @functools.partial(
        pl.kernel,
        out_type=jax.ShapeDtypeStruct((NC, NPAD, D), jnp.float32),
        mesh=_sc_mesh(),
        compiler_params=pltpu.CompilerParams(use_tc_tiling_on_sc=False),
        scratch_types=[
            pltpu.VMEM((NCHUNK, CH), jnp.int32),
            pltpu.VMEM((NCHUNK, CH), jnp.int32),
            pltpu.VMEM((NCHUNK, CH), jnp.float32),
        ]
        + [pltpu.VMEM((CH, D), jnp.float32) for _ in range(NBUF)]
        + [pltpu.VMEM_SHARED((NPAD, D), jnp.float32)]
        + [pltpu.SemaphoreType.DMA for _ in range(2 * NBUF)],
    )
    def _sc_agg(src3, dst3, ew3, tbl, zeros, out, src_all, dst_all, ew_all,
                *bufs_acc_sems):
        rows = bufs_acc_sems[:NBUF]
        acc = bufs_acc_sems[NBUF]
        gsems = bufs_acc_sems[NBUF + 1:2 * NBUF + 1]
        ssems = bufs_acc_sems[2 * NBUF + 1:]
        cid = lax.axis_index("c")
        sid = lax.axis_index("s")
        wid = cid * NS + sid
        pltpu.sync_copy(src3.at[wid], src_all)
        pltpu.sync_copy(dst3.at[wid], dst_all)
        pltpu.sync_copy(ew3.at[wid], ew_all)
        pltpu.sync_copy(zeros.at[pl.ds(sid * RPT, RPT)], acc.at[pl.ds(sid * RPT, RPT)])
        plsc.subcore_barrier()

        def g_copy(j, b):
            return pltpu.make_async_copy(tbl.at[src_all.at[j]], rows[b], gsems[b])

        def s_copy(j, b):
            return pltpu.make_async_copy(rows[b], acc.at[dst_all.at[j]], ssems[b])

        def scale(b, j):
            def grp(g, c):
                ev = ew_all[j, pl.ds(g * 16, 16)]
                for i in range(16):
                    s = ev[i]
                    for k in range(D // 16):
                        sl = pl.ds(k * 16, 16)
                        rows[b][g * 16 + i, sl] = rows[b][g * 16 + i, sl] * s
                return c

            lax.fori_loop(0, CH // 16, grp, 0)

        for b in range(NBUF):
            g_copy(b, b).start()

        def step(j2, carry):
            j0 = NBUF * j2
            for b in range(NBUF):
                j = j0 + b
                g_copy(j, b).wait()
                scale(b, j)
                s_copy(j, b).start(add=True)

                @pl.when(j + NBUF < NCHUNK)
                def _():
                    s_copy(j, b).wait()
                    g_copy(j + NBUF, b).start()

            return carry

        lax.fori_loop(0, NCHUNK // NBUF, step, 0)
        for b in range(NBUF):
            s_copy(NCHUNK - NBUF + b, b).wait()
        plsc.subcore_barrier()
        pltpu.sync_copy(
            acc.at[pl.ds(sid * RPT, RPT)], out.at[cid, pl.ds(sid * RPT, RPT)]
        )

    return _sc_agg


_sc_agg64 = _make_sc_agg(64)
_sc_agg32 = _make_sc_agg(32)


def _tc1_body(d0, d1, x, w, dinv_o, xws_o):
    deg = d0[...] + d1[...] + 1.0
    dinv = lax.rsqrt(deg)
    xw = jnp.dot(x[...], w[...], preferred_element_type=jnp.float32)
    dinv_o[...] = dinv
    xws_o[...] = xw * dinv


_tc1 = pl.pallas_call(
    _tc1_body,
    out_shape=[
        jax.ShapeDtypeStruct((N, 1), jnp.float32),
        jax.ShapeDtypeStruct((N, 64), jnp.float32),
    ],
)


def _tc2_body(a0, a1, xws, dinv, b1, w2, out):
    t = (a0[...] + a1[...] + xws[...]) * dinv[...] + b1[...]
    h = jnp.maximum(t, 0.0)
    hw2 = jnp.dot(h, w2[...], preferred_element_type=jnp.float32)
    out[...] = hw2 * dinv[...]


_tc2 = pl.pallas_call(
    _tc2_body,
    out_shape=jax.ShapeDtypeStruct((N, 32), jnp.float32),
)


def _tc3_body(c0, c1, hw, dinv, b2, out):
    out[...] = (c0[...] + c1[...] + hw[...]) * dinv[...] + b2[...]


_tc3 = pl.pallas_call(
    _tc3_body,
    out_shape=jax.ShapeDtypeStruct((N, 32), jnp.float32),
)


def kernel(x, edge_index, edge_weight, W1, b1, W2, b2):
    ei = edge_index.astype(jnp.int32)
    src3 = ei[0].reshape(NW, NCHUNK, CH)
    dst3 = ei[1].reshape(NW, NCHUNK, CH)
    ew3 = edge_weight.reshape(NW, NCHUNK, CH)
    z1 = jnp.zeros((NPAD,), jnp.float32)
    z64 = jnp.zeros((NPAD, 64), jnp.float32)
    z32 = jnp.zeros((NPAD, 32), jnp.float32)

    degp = _sc_deg(dst3, ew3, z1)
    d0 = degp[0, :N, None]
    d1 = degp[1, :N, None]
    dinv, xws1 = _tc1(d0, d1, x, W1)
    acc1 = _sc_agg64(src3, dst3, ew3, xws1, z64)
    hw2s = _tc2(acc1[0, :N], acc1[1, :N], xws1, dinv,
                b1.reshape(1, -1), W2)
    acc2 = _sc_agg32(src3, dst3, ew3, hw2s, z32)
    z = _tc3(acc2[0, :N], acc2[1, :N], hw2s, dinv, b2.reshape(1, -1))
    return z
